# Initial kernel scaffold; baseline (speedup 1.0000x reference)
import jax
import jax.numpy as jnp
from jax.experimental import pallas as pl


def kernel(edge_index, basis, edge_weights, node_features):
    raise NotImplementedError("write your pallas kernel here")


# speedup vs baseline: 6.4623x; 6.4623x over previous
import functools

import jax
import jax.numpy as jnp
from jax import lax
from jax.experimental import pallas as pl
from jax.experimental.pallas import tpu as pltpu
from jax.experimental.pallas import tpu_sc as plsc

N_EDGES = 1600000
NUM_WORKERS = 32
EDGES_PER_WORKER = N_EDGES // NUM_WORKERS
CHUNK = 400
CHUNKS_PER_WORKER = EDGES_PER_WORKER // CHUNK
IDX_MINOR = 100
IDX_ROWS = CHUNK // IDX_MINOR
GROUPS = CHUNK // 16


def _col_of(m, i):
    return m if i == 0 else 3 * m + 3 + i


def _outcol(n, o):
    return n if o == 0 else 3 * n + 3 + o


def _make_sc_kernel():
    mesh = plsc.VectorSubcoreMesh(core_axis_name="c", subcore_axis_name="s")

    @functools.partial(
        pl.kernel,
        mesh=mesh,
        compiler_params=pltpu.CompilerParams(
            needs_layout_passes=False, use_tc_tiling_on_sc=False
        ),
        out_type=jax.ShapeDtypeStruct((N_EDGES, 16), jnp.float32),
        scratch_types=[
            pltpu.VMEM((IDX_ROWS, IDX_MINOR), jnp.int32),
            pltpu.VMEM((CHUNK, 16), jnp.float32),
            pltpu.VMEM((CHUNK, 16), jnp.float32),
            pltpu.VMEM((CHUNK, 16), jnp.float32),
            pltpu.VMEM((CHUNK, 16), jnp.float32),
            pltpu.SemaphoreType.DMA,
        ],
    )
    def sc_kernel(u_hbm, b_hbm, w_hbm, nf_hbm, out_hbm,
                  idx_v, f_v, b_v, w_v, o_v, sem):
        wid = lax.axis_index("s") * 2 + lax.axis_index("c")
        iota16 = lax.iota(jnp.int32, 16)

        def chunk_body(c, carry):
            ebase = wid * EDGES_PER_WORKER + c * CHUNK
            rbase = wid * (EDGES_PER_WORKER // IDX_MINOR) + c * IDX_ROWS
            pltpu.sync_copy(u_hbm.at[pl.ds(rbase, IDX_ROWS)], idx_v)
            handles = [
                pltpu.async_copy(
                    nf_hbm.at[idx_v.at[j]],
                    f_v.at[pl.ds(j * IDX_MINOR, IDX_MINOR)],
                    sem,
                )
                for j in range(IDX_ROWS)
            ]
            pltpu.sync_copy(b_hbm.at[pl.ds(ebase, CHUNK)], b_v)
            pltpu.sync_copy(w_hbm.at[pl.ds(ebase, CHUNK)], w_v)
            for h in handles:
                h.wait()

            def group(g, gcarry):
                eidx = g * 16 + iota16

                def gcol(ref, k):
                    kvec = jnp.full((16,), k, jnp.int32)
                    return plsc.load_gather(ref, [eidx, kvec])

                fc = [gcol(f_v, k) for k in range(16)]
                bc = [gcol(b_v, k) for k in range(16)]
                tmp = []
                for m in range(4):
                    row = []
                    for o in range(4):
                        acc = fc[_col_of(m, 0)] * bc[o]
                        for i in range(1, 4):
                            acc = acc + fc[_col_of(m, i)] * bc[i * 4 + o]
                        row.append(acc)
                    tmp.append(row)
                wc = [gcol(w_v, k) for k in range(16)]
                for n in range(4):
                    for o in range(4):
                        acc = wc[n * 4] * tmp[0][o]
                        for m in range(1, 4):
                            acc = acc + wc[n * 4 + m] * tmp[m][o]
                        kvec = jnp.full((16,), _outcol(n, o), jnp.int32)
                        plsc.store_scatter(o_v, [eidx, kvec], acc)
                return gcarry

            lax.fori_loop(0, GROUPS, group, 0)
            pltpu.sync_copy(o_v, out_hbm.at[pl.ds(ebase, CHUNK)])
            return carry

        lax.fori_loop(0, CHUNKS_PER_WORKER, chunk_body, 0)

    return sc_kernel


_SC_KERNEL = _make_sc_kernel()


def kernel(edge_index, basis, edge_weights, node_features):
    e = basis.shape[0]
    u2d = edge_index[0].reshape(e // IDX_MINOR, IDX_MINOR)
    b2 = basis.reshape(e, 16)
    w2 = edge_weights.reshape(e, 16)
    return _SC_KERNEL(u2d, b2, w2, node_features)


# speedup vs baseline: 7.3168x; 1.1322x over previous
import functools

import jax
import jax.numpy as jnp
from jax import lax
from jax.experimental import pallas as pl
from jax.experimental.pallas import tpu as pltpu
from jax.experimental.pallas import tpu_sc as plsc

N_EDGES = 1600000
NUM_WORKERS = 32
EDGES_PER_WORKER = N_EDGES // NUM_WORKERS
CHUNK = 400
NCHUNKS = EDGES_PER_WORKER // CHUNK
IDX_MINOR = 100
IDX_ROWS = CHUNK // IDX_MINOR
GROUPS = CHUNK // 16


def _col_of(m, i):
    return m if i == 0 else 3 * m + 3 + i


def _outcol(n, o):
    return n if o == 0 else 3 * n + 3 + o


def _make_sc_kernel():
    mesh = plsc.VectorSubcoreMesh(core_axis_name="c", subcore_axis_name="s")

    vm = pltpu.VMEM
    scratch = (
        [vm((IDX_ROWS, IDX_MINOR), jnp.int32) for _ in range(2)]
        + [vm((CHUNK, 16), jnp.float32) for _ in range(8)]
        + [pltpu.SemaphoreType.DMA for _ in range(10)]
    )

    @functools.partial(
        pl.kernel,
        mesh=mesh,
        compiler_params=pltpu.CompilerParams(
            needs_layout_passes=False, use_tc_tiling_on_sc=False
        ),
        out_type=jax.ShapeDtypeStruct((N_EDGES, 16), jnp.float32),
        scratch_types=scratch,
    )
    def sc_kernel(u_hbm, b_hbm, w_hbm, nf_hbm, out_hbm,
                  idx0, idx1, f0, f1, b0, b1, w0, w1, o0, o1,
                  si0, si1, sg0, sg1, sb0, sb1, sw0, sw1, so0, so1):
        wid = lax.axis_index("s") * 2 + lax.axis_index("c")
        iota16 = lax.iota(jnp.int32, 16)
        idx_v = (idx0, idx1)
        f_v = (f0, f1)
        b_v = (b0, b1)
        w_v = (w0, w1)
        o_v = (o0, o1)
        sem_i = (si0, si1)
        sem_g = (sg0, sg1)
        sem_b = (sb0, sb1)
        sem_w = (sw0, sw1)
        sem_o = (so0, so1)

        def ebase(c):
            return wid * EDGES_PER_WORKER + c * CHUNK

        def rbase(c):
            return wid * (EDGES_PER_WORKER // IDX_MINOR) + c * IDX_ROWS

        def idx_desc(c, p):
            return pltpu.make_async_copy(
                u_hbm.at[pl.ds(rbase(c), IDX_ROWS)], idx_v[p], sem_i[p]
            )

        def gather_descs(c, p):
            return [
                pltpu.make_async_copy(
                    nf_hbm.at[idx_v[p].at[j]],
                    f_v[p].at[pl.ds(j * IDX_MINOR, IDX_MINOR)],
                    sem_g[p],
                )
                for j in range(IDX_ROWS)
            ]

        def b_desc(c, p):
            return pltpu.make_async_copy(
                b_hbm.at[pl.ds(ebase(c), CHUNK)], b_v[p], sem_b[p]
            )

        def w_desc(c, p):
            return pltpu.make_async_copy(
                w_hbm.at[pl.ds(ebase(c), CHUNK)], w_v[p], sem_w[p]
            )

        def out_desc(c, p):
            return pltpu.make_async_copy(
                o_v[p], out_hbm.at[pl.ds(ebase(c), CHUNK)], sem_o[p]
            )

        def issue_inputs(c, p):
            for d in gather_descs(c, p):
                d.start()
            b_desc(c, p).start()
            w_desc(c, p).start()

        def wait_inputs(c, p):
            for d in gather_descs(c, p):
                d.wait()
            b_desc(c, p).wait()
            w_desc(c, p).wait()

        def compute(c, p):
            fp, bp, wp, op = f_v[p], b_v[p], w_v[p], o_v[p]

            def group(g, gcarry):
                eidx = g * 16 + iota16

                def gcol(ref, k):
                    kvec = jnp.full((16,), k, jnp.int32)
                    return plsc.load_gather(ref, [eidx, kvec])

                fc = [gcol(fp, k) for k in range(16)]
                bc = [gcol(bp, k) for k in range(16)]
                tmp = []
                for m in range(4):
                    row = []
                    for o in range(4):
                        acc = fc[_col_of(m, 0)] * bc[o]
                        for i in range(1, 4):
                            acc = acc + fc[_col_of(m, i)] * bc[i * 4 + o]
                        row.append(acc)
                    tmp.append(row)
                wc = [gcol(wp, k) for k in range(16)]
                for n in range(4):
                    for o in range(4):
                        acc = wc[n * 4] * tmp[0][o]
                        for m in range(1, 4):
                            acc = acc + wc[n * 4 + m] * tmp[m][o]
                        kvec = jnp.full((16,), _outcol(n, o), jnp.int32)
                        plsc.store_scatter(op, [eidx, kvec], acc)
                return gcarry

            lax.fori_loop(0, GROUPS, group, 0)

        def process(c, p):
            q = 1 - p
            @pl.when(c + 1 < NCHUNKS)
            def _():
                idx_desc(c + 1, q).wait()
                issue_inputs(c + 1, q)

            wait_inputs(c, p)

            @pl.when(c + 2 < NCHUNKS)
            def _():
                idx_desc(c + 2, p).start()

            @pl.when(c >= 2)
            def _():
                out_desc(c - 2, p).wait()

            compute(c, p)
            out_desc(c, p).start()

        idx_desc(0, 0).start()
        idx_desc(0, 0).wait()
        issue_inputs(0, 0)
        idx_desc(1, 1).start()

        @pl.loop(0, NCHUNKS - 1, step=2)
        def _(c):
            process(c, 0)
            process(c + 1, 1)

        process(jnp.int32(NCHUNKS - 1), 0)

        out_desc(NCHUNKS - 2, 1).wait()
        out_desc(NCHUNKS - 1, 0).wait()

    return sc_kernel


_SC_KERNEL = _make_sc_kernel()


def kernel(edge_index, basis, edge_weights, node_features):
    e = basis.shape[0]
    u2d = edge_index[0].reshape(e // IDX_MINOR, IDX_MINOR)
    b2 = basis.reshape(e, 16)
    w2 = edge_weights.reshape(e, 16)
    return _SC_KERNEL(u2d, b2, w2, node_features)
